# Initial kernel scaffold; baseline (speedup 1.0000x reference)
import jax
import jax.numpy as jnp
from jax.experimental import pallas as pl


def kernel(x, adj_indices, adj_values, params):
    raise NotImplementedError("write your pallas kernel here")


# speedup vs baseline: 17.8603x; 17.8603x over previous
import functools

import jax
import jax.numpy as jnp
from jax import lax
from jax.experimental import pallas as pl
from jax.experimental.pallas import tpu as pltpu

N_V = 1723
NP = 1792
EPS = 1e-5


def _col_mask(nrows):
    it = lax.broadcasted_iota(jnp.int32, (1, NP), 1)
    return (it < N_V).astype(jnp.float32)


def _gn_relu(h, g, b, num_groups, mask):
    C = h.shape[0]
    gs = C // num_groups
    r0 = lax.broadcasted_iota(jnp.int32, (num_groups, C), 0)
    c0 = lax.broadcasted_iota(jnp.int32, (num_groups, C), 1)
    G = (c0 // gs == r0).astype(jnp.float32)
    r1 = lax.broadcasted_iota(jnp.int32, (C, num_groups), 0)
    c1 = lax.broadcasted_iota(jnp.int32, (C, num_groups), 1)
    GT = (r1 // gs == c1).astype(jnp.float32)
    cnt = float(gs * N_V)
    s1 = jnp.sum(h, axis=1, keepdims=True)
    s2 = jnp.sum(h * h, axis=1, keepdims=True)
    gm = jnp.dot(G, s1, preferred_element_type=jnp.float32) / cnt
    gm2 = jnp.dot(G, s2, preferred_element_type=jnp.float32) / cnt
    inv = lax.rsqrt(jnp.maximum(gm2 - gm * gm, 0.0) + EPS)
    mean_c = jnp.dot(GT, gm, preferred_element_type=jnp.float32)
    inv_c = jnp.dot(GT, inv, preferred_element_type=jnp.float32)
    xn = (h - mean_c) * inv_c
    return jnp.maximum(xn * g + b, 0.0) * mask


def _mm(W, x):
    return jnp.dot(W, x, preferred_element_type=jnp.float32)


def _gc0_body(x_ref, w_ref, b_ref, o_ref):
    j = pl.program_id(1)
    xb = x_ref[0]
    y = _mm(w_ref[...], xb) + b_ref[...]
    col = lax.broadcasted_iota(jnp.int32, y.shape, 1) + j * xb.shape[1]
    o_ref[0] = y * (col < N_V).astype(jnp.float32)


def _gc0(xp, Wp, bp):
    B, CK, _ = xp.shape
    CO = Wp.shape[0]
    NB = 256
    return pl.pallas_call(
        _gc0_body,
        grid=(B, NP // NB),
        in_specs=[
            pl.BlockSpec((1, CK, NB), lambda i, j: (i, 0, j)),
            pl.BlockSpec((CO, CK), lambda i, j: (0, 0)),
            pl.BlockSpec((CO, 1), lambda i, j: (0, 0)),
        ],
        out_specs=pl.BlockSpec((1, CO, NB), lambda i, j: (i, 0, j)),
        out_shape=jax.ShapeDtypeStruct((B, CO, NP), jnp.float32),
    )(xp, Wp, bp)


def _block_body(has_skip, in_c, half, out_c, *refs):
    if has_skip:
        (h_ref, B_ref, pg, pb, w1, b1, g1, bb1, cwt, cb, g2, bb2, w2, b2,
         sw, sb, o_ref) = refs
    else:
        (h_ref, B_ref, pg, pb, w1, b1, g1, bb1, cwt, cb, g2, bb2, w2, b2,
         o_ref) = refs
    mask = _col_mask(1)
    h = h_ref[0]
    y = _gn_relu(h, pg[...], pb[...], in_c // 8, mask)
    y1 = (_mm(w1[...], y) + b1[...]) * mask
    y1 = _gn_relu(y1, g1[...], bb1[...], half // 8, mask)
    s_cm = _mm(cwt[...], y1)
    z = (jnp.dot(s_cm, B_ref[...], preferred_element_type=jnp.float32)
         + cb[...]) * mask
    y2 = _gn_relu(z, g2[...], bb2[...], half // 8, mask)
    y3 = _mm(w2[...], y2) + b2[...]
    if has_skip:
        hs = _mm(sw[...], h) + sb[...]
    else:
        hs = h
    o_ref[0] = (hs + y3) * mask


def _res_block(h, Bmat, p):
    B, in_c, _ = h.shape
    half = p['conv_W'].shape[0]
    out_c = p['lin2_W'].shape[0]
    has_skip = 'skip_W' in p

    def v2(a):
        return a.reshape(-1, 1)

    ops = [h, Bmat,
           v2(p['pre_norm_g']), v2(p['pre_norm_b']),
           p['lin1_W'], v2(p['lin1_b']),
           v2(p['norm1_g']), v2(p['norm1_b']),
           p['conv_W'].T, v2(p['conv_b']),
           v2(p['norm2_g']), v2(p['norm2_b']),
           p['lin2_W'], v2(p['lin2_b'])]
    if has_skip:
        ops += [p['skip_W'], v2(p['skip_b'])]

    specs = [pl.BlockSpec((1, in_c, NP), lambda i: (i, 0, 0)),
             pl.BlockSpec((NP, NP), lambda i: (0, 0))]
    specs += [pl.BlockSpec(a.shape, lambda i: tuple(0 for _ in a.shape))
              for a in ops[2:]]

    return pl.pallas_call(
        functools.partial(_block_body, has_skip, in_c, half, out_c),
        grid=(B,),
        in_specs=specs,
        out_specs=pl.BlockSpec((1, out_c, NP), lambda i: (i, 0, 0)),
        out_shape=jax.ShapeDtypeStruct((B, out_c, NP), jnp.float32),
    )(*ops)


def _heads_body(s_ref, h_ref, sg, sb, slw, slb, cg, cb, cgw, cgb, clw, clb,
                shp_ref, cam_ref):
    mask = _col_mask(1)
    s = s_ref[0]
    sn = _gn_relu(s, sg[...], sb[...], 4, mask)
    shp_ref[0] = _mm(slw[...], sn) + slb[...]
    h = h_ref[0]
    hn = _gn_relu(h, cg[...], cb[...], 64, mask)
    c = jnp.maximum(_mm(cgw[...], hn) + cgb[...], 0.0) * mask
    cam_ref[0] = (jnp.dot(c, clw[...], preferred_element_type=jnp.float32)
                  + clb[...])


def _heads(s, h, params):
    B = s.shape[0]

    def v2(a):
        return a.reshape(-1, 1)

    slw = jnp.zeros((8, 32), jnp.float32).at[:3].set(params['shape_lin_W'])
    slb = jnp.zeros((8, 1), jnp.float32).at[:3, 0].set(params['shape_lin_b'])
    cgw = jnp.zeros((8, 512), jnp.float32).at[:1].set(params['cam_glin_W'])
    cgb = jnp.zeros((8, 1), jnp.float32).at[:1, 0].set(params['cam_glin_b'])
    clw = jnp.zeros((NP, 128), jnp.float32).at[:N_V, :3].set(
        params['cam_lin_W'].T)
    clb = jnp.zeros((1, 128), jnp.float32).at[0, :3].set(params['cam_lin_b'])

    ops = [s, h, v2(params['shape_gn_g']), v2(params['shape_gn_b']),
           slw, slb, v2(params['cam_gn_g']), v2(params['cam_gn_b']),
           cgw, cgb, clw, clb]
    specs = [pl.BlockSpec((1, 32, NP), lambda i: (i, 0, 0)),
             pl.BlockSpec((1, 512, NP), lambda i: (i, 0, 0))]
    specs += [pl.BlockSpec(a.shape, lambda i: tuple(0 for _ in a.shape))
              for a in ops[2:]]

    return pl.pallas_call(
        _heads_body,
        grid=(B,),
        in_specs=specs,
        out_specs=[pl.BlockSpec((1, 8, NP), lambda i: (i, 0, 0)),
                   pl.BlockSpec((1, 8, 128), lambda i: (i, 0, 0))],
        out_shape=[jax.ShapeDtypeStruct((B, 8, NP), jnp.float32),
                   jax.ShapeDtypeStruct((B, 8, 128), jnp.float32)],
    )(*ops)


def _build_B(adj_indices, adj_values):
    rows = adj_indices[0]
    cols = adj_indices[1]
    return jnp.zeros((NP, NP), jnp.float32).at[cols, rows].add(adj_values)


def kernel(x, adj_indices, adj_values, params):
    Bsz = x.shape[0]
    Bmat = _build_B(adj_indices, adj_values)

    xp = jnp.pad(x, ((0, 0), (0, 2056 - x.shape[1]), (0, NP - N_V)))
    Wp = jnp.pad(params['gc0_W'], ((0, 0), (0, 2056 - x.shape[1])))
    h = _gc0(xp, Wp, params['gc0_b'].reshape(-1, 1))

    for p in params['gc_blocks']:
        h = _res_block(h, Bmat, p)
    s = h
    for p in params['shape_blocks']:
        s = _res_block(s, Bmat, p)

    shp, cam = _heads(s, h, params)
    shape = shp[:, :3, :N_V]
    camera = cam[:, 0, :3]
    return (shape, camera)
